# baseline (speedup 1.0000x reference)
import functools

import jax
import jax.numpy as jnp
from jax import lax
from jax.experimental import pallas as pl
from jax.experimental.pallas import tpu as pltpu
from jax.experimental.pallas import tpu_sc as plsc

_NUM_MOD = 2
_D = 768
_E = 16
_K = 12
_H = _D // 4
_OUT = 101
_N = 8192
_EH = _E * _H

_BN = 512
_ROWS = _NUM_MOD * _N
_NW = 32
_RPW = _ROWS // _NW


def _logits_body(x_ref, wg_ref, out_ref):
    out_ref[0] = jax.lax.dot_general(x_ref[0], wg_ref[0],
                                     (((1,), (1,)), ((), ())))


def _gate_sc_body(logits_hbm, gates_hbm, lg_v, out_v):
    wid = lax.axis_index("s") * 2 + lax.axis_index("c")
    base = wid * _RPW
    pltpu.sync_copy(logits_hbm.at[pl.ds(base, _RPW)], lg_v)

    lane = lax.iota(jnp.int32, _E)

    def row(i, carry):
        v = lg_v[i, :]
        svals, sidx = plsc.sort_key_val(v, lane, descending=True)
        m = jnp.max(v)
        ex = jnp.where(lane < _K, jnp.exp(svals - m), 0.0)
        g = ex / jnp.sum(ex)
        rowidx = jnp.broadcast_to(i, (_E,)).astype(jnp.int32)
        plsc.store_scatter(out_v, [rowidx, sidx], g)
        return carry

    lax.fori_loop(0, _RPW, row, 0)
    pltpu.sync_copy(out_v, gates_hbm.at[pl.ds(base, _RPW)])


def _moe_body(x_ref, gates_ref, w1_ref, w2_ref, wout_ref, exp_ref, out_ref):
    x = x_ref[0]
    gates = gates_ref[0]
    gexp = jnp.dot(gates.astype(jnp.bfloat16), exp_ref[...],
                   preferred_element_type=jnp.float32)

    xb = x.astype(jnp.bfloat16)
    h = jnp.maximum(
        jnp.dot(xb, w1_ref[0], preferred_element_type=jnp.float32), 0.0)
    gh = (h * gexp).astype(jnp.bfloat16)
    moe = jnp.dot(gh, w2_ref[0], preferred_element_type=jnp.float32)

    xr = jnp.maximum(moe, 0.0) + x
    out_ref[0] = jnp.dot(xr.astype(jnp.bfloat16), wout_ref[0],
                         preferred_element_type=jnp.float32)


@jax.jit
def kernel(x_0, x_1, w_gate_0, W1_0, b1_0, W2_0, b2_0, Wout_0, bout_0,
           w_gate_1, W1_1, b1_1, W2_1, b2_1, Wout_1, bout_1):
    x = jnp.stack([x_0, x_1])
    wg = jnp.stack([w_gate_0, w_gate_1])
    w1 = jnp.stack([W1_0, W1_1])
    w1 = jnp.transpose(w1, (0, 2, 1, 3)).reshape(_NUM_MOD, _D, _EH)
    w1 = w1.astype(jnp.bfloat16)
    w2 = jnp.stack([W2_0, W2_1]).reshape(_NUM_MOD, _EH, _D)
    w2 = w2.astype(jnp.bfloat16)
    wout = jnp.stack([Wout_0, Wout_1]).astype(jnp.bfloat16)
    expand = jnp.repeat(jnp.eye(_E, dtype=jnp.bfloat16), _H, axis=1)

    logits = pl.pallas_call(
        _logits_body,
        grid=(_NUM_MOD, _N // 2048),
        in_specs=[
            pl.BlockSpec((1, 2048, _D), lambda m, n: (m, n, 0)),
            pl.BlockSpec((1, _E, _D), lambda m, n: (m, 0, 0)),
        ],
        out_specs=pl.BlockSpec((1, 2048, _E), lambda m, n: (m, n, 0)),
        out_shape=jax.ShapeDtypeStruct((_NUM_MOD, _N, _E), jnp.float32),
    )(x, jnp.transpose(wg, (0, 2, 1)))

    mesh = plsc.VectorSubcoreMesh(core_axis_name="c", subcore_axis_name="s")
    gate_k = functools.partial(
        pl.kernel, mesh=mesh,
        out_type=jax.ShapeDtypeStruct((_ROWS, _E), jnp.float32),
        scratch_types=[
            pltpu.VMEM((_RPW, _E), jnp.float32),
            pltpu.VMEM((_RPW, _E), jnp.float32),
        ],
        compiler_params=pltpu.CompilerParams(needs_layout_passes=False),
    )(_gate_sc_body)
    gates = gate_k(logits.reshape(_ROWS, _E)).reshape(_NUM_MOD, _N, _E)

    out = pl.pallas_call(
        _moe_body,
        grid=(_NUM_MOD, _N // _BN),
        in_specs=[
            pl.BlockSpec((1, _BN, _D), lambda m, n: (m, n, 0)),
            pl.BlockSpec((1, _BN, _E), lambda m, n: (m, n, 0)),
            pl.BlockSpec((1, _D, _EH), lambda m, n: (m, 0, 0)),
            pl.BlockSpec((1, _EH, _D), lambda m, n: (m, 0, 0)),
            pl.BlockSpec((1, _D, _OUT), lambda m, n: (m, 0, 0)),
            pl.BlockSpec((_E, _EH), lambda m, n: (0, 0)),
        ],
        out_specs=pl.BlockSpec((1, _BN, _OUT), lambda m, n: (m, n, 0)),
        out_shape=jax.ShapeDtypeStruct((_NUM_MOD, _N, _OUT), jnp.float32),
    )(x, gates, w1, w2, wout, expand)
    return out


# speedup vs baseline: 1.0014x; 1.0014x over previous
import functools

import jax
import jax.numpy as jnp
from jax import lax
from jax.experimental import pallas as pl
from jax.experimental.pallas import tpu as pltpu
from jax.experimental.pallas import tpu_sc as plsc

_NUM_MOD = 2
_D = 768
_E = 16
_K = 12
_H = _D // 4
_OUT = 101
_N = 8192
_EH = _E * _H

_BN = 512
_ROWS = _NUM_MOD * _N
_NW = 32
_RPW = _ROWS // _NW


def _logits_body(x_ref, wg_ref, out_ref):
    out_ref[0] = jax.lax.dot_general(x_ref[0], wg_ref[0],
                                     (((1,), (1,)), ((), ())))


def _gate_sc_body(logits_hbm, gates_hbm, lg_v, out_v):
    wid = lax.axis_index("s") * 2 + lax.axis_index("c")
    base = wid * _RPW
    pltpu.sync_copy(logits_hbm.at[pl.ds(base, _RPW)], lg_v)

    lane = lax.iota(jnp.int32, _E)

    def row(i, carry):
        v = lg_v[i, :]
        svals, sidx = plsc.sort_key_val(v, lane, descending=True)
        m = jnp.max(v)
        ex = jnp.where(lane < _K, jnp.exp(svals - m), 0.0)
        g = ex / jnp.sum(ex)
        rowidx = jnp.broadcast_to(i, (_E,)).astype(jnp.int32)
        plsc.store_scatter(out_v, [rowidx, sidx], g)
        return carry

    lax.fori_loop(0, _RPW, row, 0, unroll=8)
    pltpu.sync_copy(out_v, gates_hbm.at[pl.ds(base, _RPW)])


def _moe_body(x_ref, gates_ref, w1_ref, w2_ref, wout_ref, exp_ref, out_ref):
    x = x_ref[0]
    gates = gates_ref[0]
    gexp = jnp.dot(gates.astype(jnp.bfloat16), exp_ref[...],
                   preferred_element_type=jnp.float32)

    xb = x.astype(jnp.bfloat16)
    h = jnp.maximum(
        jnp.dot(xb, w1_ref[0], preferred_element_type=jnp.float32), 0.0)
    gh = (h * gexp).astype(jnp.bfloat16)
    moe = jnp.dot(gh, w2_ref[0], preferred_element_type=jnp.float32)

    xr = jnp.maximum(moe, 0.0) + x
    out_ref[0] = jnp.dot(xr.astype(jnp.bfloat16), wout_ref[0],
                         preferred_element_type=jnp.float32)


@jax.jit
def kernel(x_0, x_1, w_gate_0, W1_0, b1_0, W2_0, b2_0, Wout_0, bout_0,
           w_gate_1, W1_1, b1_1, W2_1, b2_1, Wout_1, bout_1):
    x = jnp.stack([x_0, x_1])
    wg = jnp.stack([w_gate_0, w_gate_1])
    w1 = jnp.stack([W1_0, W1_1])
    w1 = jnp.transpose(w1, (0, 2, 1, 3)).reshape(_NUM_MOD, _D, _EH)
    w1 = w1.astype(jnp.bfloat16)
    w2 = jnp.stack([W2_0, W2_1]).reshape(_NUM_MOD, _EH, _D)
    w2 = w2.astype(jnp.bfloat16)
    wout = jnp.stack([Wout_0, Wout_1]).astype(jnp.bfloat16)
    expand = jnp.repeat(jnp.eye(_E, dtype=jnp.bfloat16), _H, axis=1)

    logits = pl.pallas_call(
        _logits_body,
        grid=(_NUM_MOD, _N // 2048),
        in_specs=[
            pl.BlockSpec((1, 2048, _D), lambda m, n: (m, n, 0)),
            pl.BlockSpec((1, _E, _D), lambda m, n: (m, 0, 0)),
        ],
        out_specs=pl.BlockSpec((1, 2048, _E), lambda m, n: (m, n, 0)),
        out_shape=jax.ShapeDtypeStruct((_NUM_MOD, _N, _E), jnp.float32),
    )(x, jnp.transpose(wg, (0, 2, 1)))

    mesh = plsc.VectorSubcoreMesh(core_axis_name="c", subcore_axis_name="s")
    gate_k = functools.partial(
        pl.kernel, mesh=mesh,
        out_type=jax.ShapeDtypeStruct((_ROWS, _E), jnp.float32),
        scratch_types=[
            pltpu.VMEM((_RPW, _E), jnp.float32),
            pltpu.VMEM((_RPW, _E), jnp.float32),
        ],
        compiler_params=pltpu.CompilerParams(needs_layout_passes=False),
    )(_gate_sc_body)
    gates = gate_k(logits.reshape(_ROWS, _E)).reshape(_NUM_MOD, _N, _E)

    out = pl.pallas_call(
        _moe_body,
        grid=(_NUM_MOD, _N // _BN),
        in_specs=[
            pl.BlockSpec((1, _BN, _D), lambda m, n: (m, n, 0)),
            pl.BlockSpec((1, _BN, _E), lambda m, n: (m, n, 0)),
            pl.BlockSpec((1, _D, _EH), lambda m, n: (m, 0, 0)),
            pl.BlockSpec((1, _EH, _D), lambda m, n: (m, 0, 0)),
            pl.BlockSpec((1, _D, _OUT), lambda m, n: (m, 0, 0)),
            pl.BlockSpec((_E, _EH), lambda m, n: (0, 0)),
        ],
        out_specs=pl.BlockSpec((1, _BN, _OUT), lambda m, n: (m, n, 0)),
        out_shape=jax.ShapeDtypeStruct((_NUM_MOD, _N, _OUT), jnp.float32),
    )(x, gates, w1, w2, wout, expand)
    return out
